# baseline (speedup 1.0000x reference)
import functools

import jax
import jax.numpy as jnp
from jax import lax
from jax.experimental import pallas as pl
from jax.experimental.pallas import tpu as pltpu
from jax.experimental.pallas import tpu_sc as plsc

_TOPK = 4
_E = 64
_LANES = 16


def _scores_block(x_ref, wt_ref, b_ref, out_ref):
    logits = jnp.dot(x_ref[...], wt_ref[...],
                     preferred_element_type=jnp.float32) + b_ref[...]
    m = jnp.max(logits, axis=-1, keepdims=True)
    ex = jnp.exp(logits - m)
    out_ref[...] = ex / jnp.sum(ex, axis=-1, keepdims=True)


def _tc_scores(flat, wt, bias2, blk=1024):
    n, h = flat.shape
    e = wt.shape[1]
    return pl.pallas_call(
        _scores_block,
        grid=(n // blk,),
        in_specs=[
            pl.BlockSpec((blk, h), lambda i: (i, 0)),
            pl.BlockSpec((h, e), lambda i: (0, 0)),
            pl.BlockSpec((1, e), lambda i: (0, 0)),
        ],
        out_specs=pl.BlockSpec((blk, e), lambda i: (i, 0)),
        out_shape=jax.ShapeDtypeStruct((n, e), jnp.float32),
    )(flat, wt, bias2)


def _sc_topk_body(scores_hbm, full_hbm, idx_hbm, buf, idxbuf, *, rpw):
    nc = 2
    wid = lax.axis_index("s") * nc + lax.axis_index("c")
    base = wid * rpw
    pltpu.sync_copy(scores_hbm.at[pl.ds(base, rpw)], buf)

    iota16 = lax.broadcasted_iota(jnp.int32, (_LANES,), 0)
    ninf = jnp.float32(-jnp.inf)
    perms = [jnp.arange(_LANES, dtype=jnp.int32) ^ d for d in (8, 4, 2, 1)]

    dnums = lax.GatherDimensionNumbers(
        offset_dims=(), collapsed_slice_dims=(0,), start_index_map=(0,))

    def _shuf(x, perm):
        return lax.gather(x, perm[:, None], dnums, slice_sizes=(1,),
                          mode=lax.GatherScatterMode.PROMISE_IN_BOUNDS)

    def row_body(r, idxacc):
        v = [buf[r, pl.ds(_LANES * j, _LANES)] for j in range(4)]
        f = [jnp.zeros((_LANES,), jnp.float32) for _ in range(4)]
        for k in range(_TOPK):
            mx = jnp.maximum(jnp.maximum(v[0], v[1]),
                             jnp.maximum(v[2], v[3]))
            for p in perms:
                mx = jnp.maximum(mx, _shuf(mx, p))
            idx = jnp.full((_LANES,), _E, jnp.int32)
            for j in range(4):
                cand = jnp.where(v[j] == mx, iota16 + _LANES * j, _E)
                idx = jnp.minimum(idx, cand)
            for p in perms:
                idx = jnp.minimum(idx, _shuf(idx, p))
            for j in range(4):
                hit = (iota16 + _LANES * j) == idx
                f[j] = jnp.where(hit, mx, f[j])
                v[j] = jnp.where(hit, ninf, v[j])
            idxacc = jnp.where(iota16 == ((r % 4) * _TOPK + k), idx, idxacc)
        for j in range(4):
            buf[r, pl.ds(_LANES * j, _LANES)] = f[j]

        @pl.when(r % 4 == 3)
        def _():
            idxbuf[pl.ds((r // 4) * _LANES, _LANES)] = idxacc

        return idxacc

    lax.fori_loop(0, rpw, row_body, jnp.zeros((_LANES,), jnp.int32))

    pltpu.sync_copy(buf, full_hbm.at[pl.ds(base, rpw)])
    pltpu.sync_copy(idxbuf, idx_hbm.at[pl.ds(base * _TOPK, rpw * _TOPK)])


def _sc_topk(scores):
    n = scores.shape[0]
    rpw = n // 32
    mesh = plsc.VectorSubcoreMesh(core_axis_name="c", subcore_axis_name="s")
    fn = functools.partial(
        pl.kernel,
        out_type=[
            jax.ShapeDtypeStruct((n, _E), jnp.float32),
            jax.ShapeDtypeStruct((n * _TOPK,), jnp.int32),
        ],
        mesh=mesh,
        scratch_types=[
            pltpu.VMEM((rpw, _E), jnp.float32),
            pltpu.VMEM((rpw * _TOPK,), jnp.int32),
        ],
    )(functools.partial(_sc_topk_body, rpw=rpw))
    full, idx_flat = fn(scores)
    return full, idx_flat.reshape(n, _TOPK)


@jax.jit
def kernel(hidden_states, weight, bias):
    b, s, h = hidden_states.shape
    e = weight.shape[0]
    n = b * s
    flat = hidden_states.reshape(n, h)
    wt = weight.T
    bias2 = bias.reshape(1, e)

    nchunk = 2
    cn = n // nchunk
    fulls, idxs = [], []
    for c in range(nchunk):
        scores_c = _tc_scores(flat[c * cn:(c + 1) * cn], wt, bias2)
        full_c, idx_c = _sc_topk(scores_c)
        fulls.append(full_c)
        idxs.append(idx_c)
    return (jnp.concatenate(fulls, 0), jnp.concatenate(idxs, 0))


# speedup vs baseline: 3.0502x; 3.0502x over previous
import functools

import jax
import jax.numpy as jnp
from jax.experimental import pallas as pl
from jax.experimental.pallas import tpu as pltpu

_TOPK = 4


def _router_block(x_ref, wt_ref, b_ref, full_ref, idx_ref, sc_ref, *,
                  blk, e):
    i = pl.program_id(0)

    logits = jnp.dot(x_ref[...], wt_ref[...],
                     preferred_element_type=jnp.float32) + b_ref[...]
    m = jnp.max(logits, axis=-1, keepdims=True)
    ex = jnp.exp(logits - m)
    scores_new = ex / jnp.sum(ex, axis=-1, keepdims=True)

    scores = sc_ref[(i + 1) % 2]
    iota_f = jax.lax.broadcasted_iota(jnp.int32, (blk, e), 1).astype(jnp.float32)
    work = scores
    idx_cols = []
    for _ in range(_TOPK):
        mx = jnp.max(work, axis=-1, keepdims=True)
        idx = jnp.min(jnp.where(work == mx, iota_f, float(e)),
                      axis=-1, keepdims=True)
        work = jnp.where(iota_f == idx, -jnp.inf, work)
        idx_cols.append(idx)

    sc_ref[i % 2] = scores_new
    full_ref[...] = jnp.where(work < 0.0, scores, 0.0)
    idx_ref[...] = jnp.concatenate(idx_cols, axis=1).astype(jnp.int32)


@jax.jit
def kernel(hidden_states, weight, bias):
    b, s, h = hidden_states.shape
    e = weight.shape[0]
    n = b * s
    blk = 1024
    nb = n // blk
    flat = hidden_states.reshape(n, h)
    wt = weight.T
    bias2 = bias.reshape(1, e)

    grid = (nb + 1,)
    full, idx = pl.pallas_call(
        functools.partial(_router_block, blk=blk, e=e),
        grid=grid,
        in_specs=[
            pl.BlockSpec((blk, h), lambda i: (jnp.minimum(i, nb - 1), 0)),
            pl.BlockSpec((h, e), lambda i: (0, 0)),
            pl.BlockSpec((1, e), lambda i: (0, 0)),
        ],
        out_specs=[
            pl.BlockSpec((blk, e), lambda i: (jnp.maximum(i - 1, 0), 0)),
            pl.BlockSpec((blk, _TOPK), lambda i: (jnp.maximum(i - 1, 0), 0)),
        ],
        out_shape=[
            jax.ShapeDtypeStruct((n, e), jnp.float32),
            jax.ShapeDtypeStruct((n, _TOPK), jnp.int32),
        ],
        scratch_shapes=[pltpu.VMEM((2, blk, e), jnp.float32)],
    )(flat, wt, bias2)
    return (full, idx)
